# Initial kernel scaffold; baseline (speedup 1.0000x reference)
import jax
import jax.numpy as jnp
from jax.experimental import pallas as pl


def kernel(x, router_w):
    raise NotImplementedError("write your pallas kernel here")


# speedup vs baseline: 4.4363x; 4.4363x over previous
import functools

import jax
import jax.numpy as jnp
from jax.experimental import pallas as pl
from jax.experimental.pallas import tpu as pltpu

N_TOKENS = 8192
DIM = 2048
N_EXPERTS = 64
TOPK = 2
N_GROUPS = 2
GROUP_SIZE = N_EXPERTS // N_GROUPS

BLOCK_T = 512


def _router_block(x_ref, w_ref, wts_ref, idx_ref):
    xb = x_ref[...]
    w = w_ref[...]
    logits = jax.lax.dot_general(
        xb, w, (((1,), (1,)), ((), ())), preferred_element_type=jnp.float32
    )
    b = logits.shape[0]
    m = jnp.max(logits, axis=-1, keepdims=True)
    e = jnp.exp(logits - m)
    p = e / jnp.sum(e, axis=-1, keepdims=True)

    lane = jax.lax.broadcasted_iota(jnp.int32, (b, N_EXPERTS), 1)
    neg_inf = jnp.float32(-jnp.inf)
    in_g0 = lane < GROUP_SIZE
    g0 = jnp.max(jnp.where(in_g0, p, neg_inf), axis=-1, keepdims=True)
    g1 = jnp.max(jnp.where(in_g0, neg_inf, p), axis=-1, keepdims=True)
    sel_g1 = g1 > g0
    in_sel = jnp.logical_xor(in_g0, sel_g1)
    masked = jnp.where(in_sel, p, neg_inf)

    v1 = jnp.max(masked, axis=-1, keepdims=True)
    i1 = jnp.min(
        jnp.where(masked == v1, lane, N_EXPERTS), axis=-1, keepdims=True
    )
    masked2 = jnp.where(lane == i1, neg_inf, masked)
    v2 = jnp.max(masked2, axis=-1, keepdims=True)
    i2 = jnp.min(
        jnp.where(masked2 == v2, lane, N_EXPERTS), axis=-1, keepdims=True
    )

    wts_ref[...] = jnp.concatenate([v1, v2], axis=-1)
    idx_ref[...] = jnp.concatenate([i1, i2], axis=-1)


@jax.jit
def kernel(x, router_w):
    n = x.shape[0]
    grid = (n // BLOCK_T,)
    wts, idx = pl.pallas_call(
        _router_block,
        grid=grid,
        in_specs=[
            pl.BlockSpec((BLOCK_T, DIM), lambda i: (i, 0)),
            pl.BlockSpec((N_EXPERTS, DIM), lambda i: (0, 0)),
        ],
        out_specs=[
            pl.BlockSpec((BLOCK_T, TOPK), lambda i: (i, 0)),
            pl.BlockSpec((BLOCK_T, TOPK), lambda i: (i, 0)),
        ],
        out_shape=[
            jax.ShapeDtypeStruct((n, TOPK), jnp.float32),
            jax.ShapeDtypeStruct((n, TOPK), jnp.int32),
        ],
    )(x, router_w)
    return wts, idx


# speedup vs baseline: 5.1371x; 1.1580x over previous
import functools

import jax
import jax.numpy as jnp
from jax.experimental import pallas as pl
from jax.experimental.pallas import tpu as pltpu

N_TOKENS = 8192
DIM = 2048
N_EXPERTS = 64
TOPK = 2
N_GROUPS = 2
GROUP_SIZE = N_EXPERTS // N_GROUPS

BLOCK_T = 1024


def _router_block(x_ref, w_ref, wts_ref, idx_ref):
    xb = x_ref[...]
    w = w_ref[...]
    logits = jax.lax.dot_general(
        xb, w, (((1,), (1,)), ((), ())), preferred_element_type=jnp.float32
    )
    b = logits.shape[0]
    m = jnp.max(logits, axis=-1, keepdims=True)
    e = jnp.exp(logits - m)
    p = e / jnp.sum(e, axis=-1, keepdims=True)

    lane = jax.lax.broadcasted_iota(jnp.int32, (b, N_EXPERTS), 1)
    neg_inf = jnp.float32(-jnp.inf)
    in_g0 = lane < GROUP_SIZE
    g0 = jnp.max(jnp.where(in_g0, p, neg_inf), axis=-1, keepdims=True)
    g1 = jnp.max(jnp.where(in_g0, neg_inf, p), axis=-1, keepdims=True)
    sel_g1 = g1 > g0
    in_sel = jnp.logical_xor(in_g0, sel_g1)
    masked = jnp.where(in_sel, p, neg_inf)

    v1 = jnp.max(masked, axis=-1, keepdims=True)
    i1 = jnp.min(
        jnp.where(masked == v1, lane, N_EXPERTS), axis=-1, keepdims=True
    )
    masked2 = jnp.where(lane == i1, neg_inf, masked)
    v2 = jnp.max(masked2, axis=-1, keepdims=True)
    i2 = jnp.min(
        jnp.where(masked2 == v2, lane, N_EXPERTS), axis=-1, keepdims=True
    )

    wts_ref[...] = jnp.concatenate([v1, v2], axis=-1)
    idx_ref[...] = jnp.concatenate([i1, i2], axis=-1)


@jax.jit
def kernel(x, router_w):
    n = x.shape[0]
    grid = (n // BLOCK_T,)
    wts, idx = pl.pallas_call(
        _router_block,
        grid=grid,
        in_specs=[
            pl.BlockSpec((BLOCK_T, DIM), lambda i: (i, 0)),
            pl.BlockSpec((N_EXPERTS, DIM), lambda i: (0, 0)),
        ],
        out_specs=[
            pl.BlockSpec((BLOCK_T, TOPK), lambda i: (i, 0)),
            pl.BlockSpec((BLOCK_T, TOPK), lambda i: (i, 0)),
        ],
        out_shape=[
            jax.ShapeDtypeStruct((n, TOPK), jnp.float32),
            jax.ShapeDtypeStruct((n, TOPK), jnp.int32),
        ],
    )(x, router_w)
    return wts, idx


# speedup vs baseline: 5.1895x; 1.0102x over previous
import functools

import jax
import jax.numpy as jnp
from jax.experimental import pallas as pl
from jax.experimental.pallas import tpu as pltpu

N_TOKENS = 8192
DIM = 2048
N_EXPERTS = 64
TOPK = 2
N_GROUPS = 2
GROUP_SIZE = N_EXPERTS // N_GROUPS

BLOCK_T = 2048


def _router_block(x_ref, w_ref, wts_ref, idx_ref):
    xb = x_ref[...]
    w = w_ref[...]
    logits = jax.lax.dot_general(
        xb, w, (((1,), (1,)), ((), ())), preferred_element_type=jnp.float32
    )
    b = logits.shape[0]
    m = jnp.max(logits, axis=-1, keepdims=True)
    e = jnp.exp(logits - m)
    p = e / jnp.sum(e, axis=-1, keepdims=True)

    lane = jax.lax.broadcasted_iota(jnp.int32, (b, N_EXPERTS), 1)
    neg_inf = jnp.float32(-jnp.inf)
    in_g0 = lane < GROUP_SIZE
    g0 = jnp.max(jnp.where(in_g0, p, neg_inf), axis=-1, keepdims=True)
    g1 = jnp.max(jnp.where(in_g0, neg_inf, p), axis=-1, keepdims=True)
    sel_g1 = g1 > g0
    in_sel = jnp.logical_xor(in_g0, sel_g1)
    masked = jnp.where(in_sel, p, neg_inf)

    v1 = jnp.max(masked, axis=-1, keepdims=True)
    i1 = jnp.min(
        jnp.where(masked == v1, lane, N_EXPERTS), axis=-1, keepdims=True
    )
    masked2 = jnp.where(lane == i1, neg_inf, masked)
    v2 = jnp.max(masked2, axis=-1, keepdims=True)
    i2 = jnp.min(
        jnp.where(masked2 == v2, lane, N_EXPERTS), axis=-1, keepdims=True
    )

    wts_ref[...] = jnp.concatenate([v1, v2], axis=-1)
    idx_ref[...] = jnp.concatenate([i1, i2], axis=-1)


@jax.jit
def kernel(x, router_w):
    n = x.shape[0]
    grid = (n // BLOCK_T,)
    wts, idx = pl.pallas_call(
        _router_block,
        grid=grid,
        in_specs=[
            pl.BlockSpec((BLOCK_T, DIM), lambda i: (i, 0)),
            pl.BlockSpec((N_EXPERTS, DIM), lambda i: (0, 0)),
        ],
        out_specs=[
            pl.BlockSpec((BLOCK_T, TOPK), lambda i: (i, 0)),
            pl.BlockSpec((BLOCK_T, TOPK), lambda i: (i, 0)),
        ],
        out_shape=[
            jax.ShapeDtypeStruct((n, TOPK), jnp.float32),
            jax.ShapeDtypeStruct((n, TOPK), jnp.int32),
        ],
    )(x, router_w)
    return wts, idx
